# baseline (speedup 1.0000x reference)
import functools

import jax
import jax.numpy as jnp
from jax import lax
from jax.experimental import pallas as pl
from jax.experimental.pallas import tpu as pltpu
from jax.experimental.pallas import tpu_sc as plsc

H = 4
K_EDGES = 32
NUM_TILES = 32


def _largest_div(n, cap):
    for d in range(min(n, cap), 0, -1):
        if n % d == 0:
            return d
    return 1


def _gat_edge_sc(xl, xr, idx, att, n, c):
    cp = c + 16
    nchunk = idx.shape[1]
    assert idx.shape == (NUM_TILES, nchunk, 3 * K_EDGES) and nchunk % 2 == 0
    npt = -(-(-(-n // 16)) // 128) * 128
    while 16 * npt <= n:
        npt += 128
    n_pad = 16 * npt
    zr = K_EDGES
    assert npt % zr == 0
    nz = npt // zr
    cblk = c // 16

    mesh = plsc.VectorSubcoreMesh(core_axis_name="c", subcore_axis_name="s",
                                  num_cores=2, num_subcores=16)

    @functools.partial(
        pl.kernel,
        out_type=jax.ShapeDtypeStruct((2, H, n_pad, cp), jnp.float32),
        mesh=mesh,
        scratch_types=[
            pltpu.VMEM((nchunk, 3 * K_EDGES), jnp.int32),
            [pltpu.VMEM((K_EDGES,), jnp.int32)] * 2,
            [pltpu.VMEM((K_EDGES,), jnp.int32)] * 2,
            [pltpu.VMEM((K_EDGES,), jnp.int32)] * 2,
            [pltpu.VMEM((K_EDGES, c), jnp.float32)] * 2,
            [pltpu.VMEM((K_EDGES, c), jnp.float32)] * 2,
            [pltpu.VMEM((K_EDGES, cp), jnp.float32)] * 2,
            pltpu.VMEM((c, 16), jnp.float32),
            pltpu.VMEM_SHARED((n_pad, cp), jnp.float32),
            [pltpu.SemaphoreType.DMA] * 2,
            [pltpu.SemaphoreType.DMA] * 2,
        ],
        compiler_params=pltpu.CompilerParams(needs_layout_passes=False,
                                             use_tc_tiling_on_sc=False),
    )
    def k(xl_hbm, xr_hbm, idx_hbm, att_hbm, out_hbm,
          idxall, srchv, dsthv, dscv, rl, rr, sbuf,
          attv, acc, gsem, ssem):
        core = lax.axis_index("c")
        sub = lax.axis_index("s")
        tid = core * 16 + sub
        row0 = sub * npt

        pltpu.sync_copy(idx_hbm.at[tid], idxall)

        z16 = jnp.zeros((16,), jnp.float32)

        @pl.loop(0, H)
        def _head(h):
            @pl.loop(0, K_EDGES)
            def _(i):
                for cb in range(cp // 16):
                    sbuf[0][i, pl.ds(cb * 16, 16)] = z16

            for j in range(nz):
                pltpu.sync_copy(sbuf[0], acc.at[pl.ds(row0 + j * zr, zr)])
            pltpu.sync_copy(att_hbm.at[h], attv)
            hn = h * n

            def load_idx(g, b):
                for j in range(K_EDGES // 16):
                    srchv[b][pl.ds(j * 16, 16)] = (
                        idxall[g, pl.ds(j * 16, 16)] + hn)
                    dsthv[b][pl.ds(j * 16, 16)] = (
                        idxall[g, pl.ds(K_EDGES + j * 16, 16)] + hn)

            def start_gather(b):
                pltpu.async_copy(xl_hbm.at[srchv[b]], rl[b], gsem[b])
                pltpu.async_copy(xr_hbm.at[dsthv[b]], rr[b], gsem[b])

            def wait_gather(b):
                pltpu.make_async_copy(xl_hbm.at[srchv[b]], rl[b],
                                      gsem[b]).wait()
                pltpu.make_async_copy(xr_hbm.at[dsthv[b]], rr[b],
                                      gsem[b]).wait()

            def wait_scatter(b):
                pltpu.make_async_copy(sbuf[b], acc.at[dscv[b]],
                                      ssem[b]).wait()

            def compute(g, b):
                for j in range(K_EDGES // 16):
                    dscv[b][pl.ds(j * 16, 16)] = idxall[
                        g, pl.ds(2 * K_EDGES + j * 16, 16)]
                lane = lax.iota(jnp.int32, 16)
                zero = jnp.zeros((16,), jnp.float32)
                for t in range(K_EDGES // 16):
                    rowv = lane + (t * 16)

                    def epass(jj, accs):
                        a0, a1, a2, a3 = accs
                        accs = [a0, a1, a2, a3]
                        ch0 = jj * 16
                        for j in range(16):
                            chv = jnp.full((16,), ch0 + j, jnp.int32)
                            l = plsc.load_gather(rl[b], [rowv, chv])
                            r = plsc.load_gather(rr[b], [rowv, chv])
                            s = l + r
                            lrel = jnp.maximum(s, 0.2 * s)
                            av = attv[ch0 + j]
                            accs[j % 4] = accs[j % 4] + lrel * av
                        return tuple(accs)

                    a0, a1, a2, a3 = lax.fori_loop(
                        0, cblk, epass, (zero, zero, zero, zero))
                    ex = jnp.exp((a0 + a1) + (a2 + a3))
                    plsc.store_scatter(
                        sbuf[b], [rowv, jnp.full((16,), c, jnp.int32)], ex)

                    @pl.loop(0, cblk)
                    def _scale(jj):
                        ch0 = jj * 16
                        for j in range(16):
                            chv = jnp.full((16,), ch0 + j, jnp.int32)
                            l = plsc.load_gather(rl[b], [rowv, chv])
                            plsc.store_scatter(sbuf[b], [rowv, chv], l * ex)

                pltpu.async_copy(sbuf[b], acc.at[dscv[b]], ssem[b],
                                 add=True)

            plsc.subcore_barrier()

            load_idx(0, 0)
            start_gather(0)

            @pl.loop(0, nchunk, step=2)
            def _chunk(g):
                load_idx(g + 1, 1)
                start_gather(1)
                wait_gather(0)

                @pl.when(g >= 2)
                def _():
                    wait_scatter(0)
                compute(g, 0)

                @pl.when(g + 2 < nchunk)
                def _():
                    load_idx(g + 2, 0)
                    start_gather(0)
                wait_gather(1)

                @pl.when(g >= 2)
                def _():
                    wait_scatter(1)
                compute(g + 1, 1)

            wait_scatter(0)
            wait_scatter(1)
            plsc.subcore_barrier()
            pltpu.sync_copy(acc.at[pl.ds(row0, npt)],
                            out_hbm.at[core, h, pl.ds(row0, npt)])
            plsc.subcore_barrier()

    return k(xl, xr, idx, att)


def _proj_heads(xs, wlh, wrh, c):
    n = xs[0].shape[0]
    nblk = _largest_div(n, 1024)
    f_tot = wlh.shape[1]
    splits = [x.shape[1] for x in xs]

    def body(*refs):
        xrefs = refs[:len(xs)]
        wl_ref, wr_ref, xl_ref, xr_ref = refs[len(xs):]
        for h in range(H):
            accl = None
            accr = None
            off = 0
            for xi, fi in zip(xrefs, splits):
                xb = xi[...]
                pl_w = wl_ref[h, pl.ds(off, fi), :]
                pr_w = wr_ref[h, pl.ds(off, fi), :]
                dl = jnp.dot(xb, pl_w, preferred_element_type=jnp.float32)
                dr = jnp.dot(xb, pr_w, preferred_element_type=jnp.float32)
                accl = dl if accl is None else accl + dl
                accr = dr if accr is None else accr + dr
                off += fi
            xl_ref[h] = accl
            xr_ref[h] = accr

    grid = (n // nblk,)
    in_specs = [pl.BlockSpec((nblk, fi), lambda i: (i, 0)) for fi in splits]
    in_specs += [pl.BlockSpec((H, f_tot, c), lambda i: (0, 0, 0))] * 2
    out_specs = [pl.BlockSpec((H, nblk, c), lambda i: (0, i, 0))] * 2
    out_shape = [jax.ShapeDtypeStruct((H, n, c), jnp.float32)] * 2
    xl3, xr3 = pl.pallas_call(
        body, grid=grid, in_specs=in_specs, out_specs=out_specs,
        out_shape=out_shape)(*xs, wlh, wrh)
    return xl3.reshape(H * n, c), xr3.reshape(H * n, c)


def _combine(acc, b, g, be, rm, rv, n, c):
    cp = c + 16
    nblk = _largest_div(n, 1024)

    def body(acc_ref, b_ref, g_ref, be_ref, rm_ref, rv_ref, o_ref):
        for h in range(H):
            a = acc_ref[0, h] + acc_ref[1, h]
            num = a[:, :c]
            den = a[:, c:c + 1]
            v = num / (den + 1e-16) + b_ref[h]
            v = jnp.maximum(v, 0.0)
            v = (v - rm_ref[h]) * jax.lax.rsqrt(rv_ref[h] + 1e-5)
            o_ref[:, h, :] = v * g_ref[h] + be_ref[h]

    grid = (n // nblk,)
    vec = pl.BlockSpec((H, 1, c), lambda i: (0, 0, 0))
    r3 = lambda a: a.reshape(H, 1, c)
    return pl.pallas_call(
        body, grid=grid,
        in_specs=[pl.BlockSpec((2, H, nblk, cp), lambda i: (0, 0, i, 0)),
                  vec, vec, vec, vec, vec],
        out_specs=pl.BlockSpec((nblk, H, c), lambda i: (i, 0, 0)),
        out_shape=jax.ShapeDtypeStruct((n, H, c), jnp.float32),
    )(acc, r3(b), r3(g), r3(be), r3(rm), r3(rv))


def _colmean(x):
    n, f = x.shape
    nblk = _largest_div(n, 1024)

    def body(x_ref, o_ref):
        @pl.when(pl.program_id(0) == 0)
        def _():
            o_ref[...] = jnp.zeros_like(o_ref)
        o_ref[...] += jnp.sum(x_ref[...], axis=0, keepdims=True) * (1.0 / n)

    return pl.pallas_call(
        body, grid=(n // nblk,),
        in_specs=[pl.BlockSpec((nblk, f), lambda i: (i, 0))],
        out_specs=pl.BlockSpec((1, f), lambda i: (0, 0)),
        out_shape=jax.ShapeDtypeStruct((1, f), jnp.float32))(x)


def _mlp(mx, m1, m2, m3, w1, b1, w2, b2, w3, b3):
    f0, f1, f2, f3 = mx.shape[1], m1.shape[1], m2.shape[1], m3.shape[1]

    def body(mx_r, m1_r, m2_r, m3_r, w1_r, b1_r, w2_r, b2_r, w3_r, b3_r, o):
        h = (jnp.dot(mx_r[...], w1_r[pl.ds(0, f0), :],
                     preferred_element_type=jnp.float32)
             + jnp.dot(m1_r[...], w1_r[pl.ds(f0, f1), :],
                       preferred_element_type=jnp.float32)
             + jnp.dot(m2_r[...], w1_r[pl.ds(f0 + f1, f2), :],
                       preferred_element_type=jnp.float32)
             + jnp.dot(m3_r[...], w1_r[pl.ds(f0 + f1 + f2, f3), :],
                       preferred_element_type=jnp.float32)
             + b1_r[...])
        h = jnp.maximum(h, 0.0)
        h2 = jnp.maximum(
            jnp.dot(h, w2_r[...], preferred_element_type=jnp.float32)
            + b2_r[...], 0.0)
        o[...] = (jnp.dot(h2, w3_r[...], preferred_element_type=jnp.float32)
                  + b3_r[...])

    nout = b3.shape[0]
    return pl.pallas_call(
        body,
        out_shape=jax.ShapeDtypeStruct((1, nout), jnp.float32),
    )(mx, m1, m2, m3, w1, b1.reshape(1, -1), w2, b2.reshape(1, -1),
      w3, b3.reshape(1, -1))


def _gat_layer(xs, wl, wr, att, b, g, be, rm, rv, idx, n, c):
    f_tot = wl.shape[0]
    wlh = wl.reshape(f_tot, H, c).transpose(1, 0, 2)
    wrh = wr.reshape(f_tot, H, c).transpose(1, 0, 2)
    xl, xr = _proj_heads(xs, wlh, wrh, c)
    attb = jnp.broadcast_to(att[:, :, None], (H, c, 16))
    acc = _gat_edge_sc(xl, xr, idx, attb, n, c)
    return _combine(acc, b, g, be, rm, rv, n, c).reshape(n, H * c)


def kernel(x, edge_index, Wl1, Wr1, att1, b1, Wl2, Wr2, att2, b2,
           Wl3, Wr3, att3, b3, g1, be1, rm1, rv1, g2, be2, rm2, rv2,
           g3, be3, rm3, rv3, Wm1, bm1, Wm2, bm2, Wm3, bm3):
    n = x.shape[0]
    e = edge_index.shape[1]
    loop = jnp.arange(n, dtype=jnp.int32)
    src = jnp.concatenate([edge_index[0].astype(jnp.int32), loop])
    dst = jnp.concatenate([edge_index[1].astype(jnp.int32), loop])
    etot = e + n
    quant = NUM_TILES * K_EDGES * 2
    epad = -(-etot // quant) * quant
    pad = epad - etot
    nchunk = epad // (NUM_TILES * K_EDGES)
    shp = (NUM_TILES, nchunk, K_EDGES)
    idx_p = jnp.concatenate(
        [jnp.pad(src, (0, pad)).reshape(shp),
         jnp.pad(dst, (0, pad)).reshape(shp),
         jnp.pad(dst, (0, pad), constant_values=n).reshape(shp)], axis=2)

    x1 = _gat_layer([x], Wl1, Wr1, att1, b1, g1, be1, rm1, rv1,
                    idx_p, n, 128)
    x2 = _gat_layer([x1], Wl2, Wr2, att2, b2, g2, be2, rm2, rv2,
                    idx_p, n, 64)
    x3 = _gat_layer([x, x2], Wl3, Wr3, att3, b3, g3, be3, rm3, rv3,
                    idx_p, n, 32)

    return _mlp(_colmean(x), _colmean(x1), _colmean(x2), _colmean(x3),
                Wm1, bm1, Wm2, bm2, Wm3, bm3)


# speedup vs baseline: 3.3490x; 3.3490x over previous
import functools

import jax
import jax.numpy as jnp
from jax import lax
from jax.experimental import pallas as pl
from jax.experimental.pallas import tpu as pltpu
from jax.experimental.pallas import tpu_sc as plsc

H = 4
K_EDGES = 32
NUM_TILES = 32


def _lane_take(v, idx16):
    dnums = lax.GatherDimensionNumbers(
        offset_dims=(), collapsed_slice_dims=(0,), start_index_map=(0,))
    return lax.gather(v, idx16[:, None], dnums, (1,),
                      mode=lax.GatherScatterMode.PROMISE_IN_BOUNDS)


def _largest_div(n, cap):
    for d in range(min(n, cap), 0, -1):
        if n % d == 0:
            return d
    return 1


def _gat_edge_sc(xl, xr, idx, att, n, c):
    cp = c + 16
    nchunk = idx.shape[1]
    assert idx.shape == (NUM_TILES, nchunk, 3 * K_EDGES) and nchunk % 2 == 0
    npt = -(-(-(-n // 16)) // 128) * 128
    while 16 * npt <= n:
        npt += 128
    n_pad = 16 * npt
    zr = K_EDGES
    assert npt % zr == 0
    nz = npt // zr
    cblk = c // 16

    mesh = plsc.VectorSubcoreMesh(core_axis_name="c", subcore_axis_name="s",
                                  num_cores=2, num_subcores=16)

    @functools.partial(
        pl.kernel,
        out_type=jax.ShapeDtypeStruct((2, H, n_pad, cp), jnp.float32),
        mesh=mesh,
        scratch_types=[
            pltpu.VMEM((nchunk, 3 * K_EDGES), jnp.int32),
            [pltpu.VMEM((K_EDGES,), jnp.int32)] * 2,
            [pltpu.VMEM((K_EDGES,), jnp.int32)] * 2,
            [pltpu.VMEM((K_EDGES,), jnp.int32)] * 2,
            [pltpu.VMEM((K_EDGES, c), jnp.float32)] * 2,
            [pltpu.VMEM((K_EDGES, c), jnp.float32)] * 2,
            [pltpu.VMEM((K_EDGES, cp), jnp.float32)] * 2,
            pltpu.VMEM((c,), jnp.float32),
            pltpu.VMEM_SHARED((n_pad, cp), jnp.float32),
            [pltpu.SemaphoreType.DMA] * 2,
            [pltpu.SemaphoreType.DMA] * 2,
        ],
        compiler_params=pltpu.CompilerParams(needs_layout_passes=False,
                                             use_tc_tiling_on_sc=False),
    )
    def k(xl_hbm, xr_hbm, idx_hbm, att_hbm, out_hbm,
          idxall, srchv, dsthv, dscv, rl, rr, sbuf,
          attv, acc, gsem, ssem):
        core = lax.axis_index("c")
        sub = lax.axis_index("s")
        tid = core * 16 + sub
        row0 = sub * npt

        pltpu.sync_copy(idx_hbm.at[tid], idxall)

        z16 = jnp.zeros((16,), jnp.float32)

        @pl.loop(0, H)
        def _head(h):
            @pl.loop(0, K_EDGES)
            def _(i):
                for cb in range(cp // 16):
                    sbuf[0][i, pl.ds(cb * 16, 16)] = z16

            for j in range(nz):
                pltpu.sync_copy(sbuf[0], acc.at[pl.ds(row0 + j * zr, zr)])
            pltpu.sync_copy(att_hbm.at[h], attv)
            att_b = [attv[pl.ds(cb * 16, 16)] for cb in range(cblk)]
            hn = h * n

            def load_idx(g, b):
                for j in range(K_EDGES // 16):
                    srchv[b][pl.ds(j * 16, 16)] = (
                        idxall[g, pl.ds(j * 16, 16)] + hn)
                    dsthv[b][pl.ds(j * 16, 16)] = (
                        idxall[g, pl.ds(K_EDGES + j * 16, 16)] + hn)

            def start_gather(b):
                pltpu.async_copy(xl_hbm.at[srchv[b]], rl[b], gsem[b])
                pltpu.async_copy(xr_hbm.at[dsthv[b]], rr[b], gsem[b])

            def wait_gather(b):
                pltpu.make_async_copy(xl_hbm.at[srchv[b]], rl[b],
                                      gsem[b]).wait()
                pltpu.make_async_copy(xr_hbm.at[dsthv[b]], rr[b],
                                      gsem[b]).wait()

            def wait_scatter(b):
                pltpu.make_async_copy(sbuf[b], acc.at[dscv[b]],
                                      ssem[b]).wait()

            def compute(g, b):
                for j in range(K_EDGES // 16):
                    dscv[b][pl.ds(j * 16, 16)] = idxall[
                        g, pl.ds(2 * K_EDGES + j * 16, 16)]
                lane = lax.iota(jnp.int32, 16)
                perms = [jnp.bitwise_xor(lane, kk) for kk in (8, 4, 2, 1)]
                is0 = lane == 0
                z16v = jnp.zeros((16,), jnp.float32)
                for i in range(K_EDGES):
                    acc0 = None
                    acc1 = None
                    for cb in range(cblk):
                        sl = pl.ds(cb * 16, 16)
                        s = rl[b][i, sl] + rr[b][i, sl]
                        lrel = jnp.maximum(s, 0.2 * s)
                        tv = lrel * att_b[cb]
                        if cb % 2 == 0:
                            acc0 = tv if acc0 is None else acc0 + tv
                        else:
                            acc1 = tv if acc1 is None else acc1 + tv
                    accv = acc0 if acc1 is None else acc0 + acc1
                    for p in perms:
                        accv = accv + _lane_take(accv, p)
                    exv = jnp.exp(accv)
                    sbuf[b][i, pl.ds(c, 16)] = jnp.where(is0, exv, z16v)
                    for cb in range(cblk):
                        sl = pl.ds(cb * 16, 16)
                        sbuf[b][i, sl] = rl[b][i, sl] * exv
                pltpu.async_copy(sbuf[b], acc.at[dscv[b]], ssem[b],
                                 add=True)

            plsc.subcore_barrier()

            load_idx(0, 0)
            start_gather(0)

            @pl.loop(0, nchunk, step=2)
            def _chunk(g):
                load_idx(g + 1, 1)
                start_gather(1)
                wait_gather(0)

                @pl.when(g >= 2)
                def _():
                    wait_scatter(0)
                compute(g, 0)

                @pl.when(g + 2 < nchunk)
                def _():
                    load_idx(g + 2, 0)
                    start_gather(0)
                wait_gather(1)

                @pl.when(g >= 2)
                def _():
                    wait_scatter(1)
                compute(g + 1, 1)

            wait_scatter(0)
            wait_scatter(1)
            plsc.subcore_barrier()
            pltpu.sync_copy(acc.at[pl.ds(row0, npt)],
                            out_hbm.at[core, h, pl.ds(row0, npt)])
            plsc.subcore_barrier()

    return k(xl, xr, idx, att)


def _proj_heads(xs, wlh, wrh, c):
    n = xs[0].shape[0]
    nblk = _largest_div(n, 1024)
    f_tot = wlh.shape[1]
    splits = [x.shape[1] for x in xs]

    def body(*refs):
        xrefs = refs[:len(xs)]
        wl_ref, wr_ref, xl_ref, xr_ref = refs[len(xs):]
        for h in range(H):
            accl = None
            accr = None
            off = 0
            for xi, fi in zip(xrefs, splits):
                xb = xi[...]
                pl_w = wl_ref[h, pl.ds(off, fi), :]
                pr_w = wr_ref[h, pl.ds(off, fi), :]
                dl = jnp.dot(xb, pl_w, preferred_element_type=jnp.float32)
                dr = jnp.dot(xb, pr_w, preferred_element_type=jnp.float32)
                accl = dl if accl is None else accl + dl
                accr = dr if accr is None else accr + dr
                off += fi
            xl_ref[h] = accl
            xr_ref[h] = accr

    grid = (n // nblk,)
    in_specs = [pl.BlockSpec((nblk, fi), lambda i: (i, 0)) for fi in splits]
    in_specs += [pl.BlockSpec((H, f_tot, c), lambda i: (0, 0, 0))] * 2
    out_specs = [pl.BlockSpec((H, nblk, c), lambda i: (0, i, 0))] * 2
    out_shape = [jax.ShapeDtypeStruct((H, n, c), jnp.float32)] * 2
    xl3, xr3 = pl.pallas_call(
        body, grid=grid, in_specs=in_specs, out_specs=out_specs,
        out_shape=out_shape)(*xs, wlh, wrh)
    return xl3.reshape(H * n, c), xr3.reshape(H * n, c)


def _combine(acc, b, g, be, rm, rv, n, c):
    cp = c + 16
    nblk = _largest_div(n, 1024)

    def body(acc_ref, b_ref, g_ref, be_ref, rm_ref, rv_ref, o_ref):
        for h in range(H):
            a = acc_ref[0, h] + acc_ref[1, h]
            num = a[:, :c]
            den = a[:, c:c + 1]
            v = num / (den + 1e-16) + b_ref[h]
            v = jnp.maximum(v, 0.0)
            v = (v - rm_ref[h]) * jax.lax.rsqrt(rv_ref[h] + 1e-5)
            o_ref[:, h, :] = v * g_ref[h] + be_ref[h]

    grid = (n // nblk,)
    vec = pl.BlockSpec((H, 1, c), lambda i: (0, 0, 0))
    r3 = lambda a: a.reshape(H, 1, c)
    return pl.pallas_call(
        body, grid=grid,
        in_specs=[pl.BlockSpec((2, H, nblk, cp), lambda i: (0, 0, i, 0)),
                  vec, vec, vec, vec, vec],
        out_specs=pl.BlockSpec((nblk, H, c), lambda i: (i, 0, 0)),
        out_shape=jax.ShapeDtypeStruct((n, H, c), jnp.float32),
    )(acc, r3(b), r3(g), r3(be), r3(rm), r3(rv))


def _colmean(x):
    n, f = x.shape
    nblk = _largest_div(n, 1024)

    def body(x_ref, o_ref):
        @pl.when(pl.program_id(0) == 0)
        def _():
            o_ref[...] = jnp.zeros_like(o_ref)
        o_ref[...] += jnp.sum(x_ref[...], axis=0, keepdims=True) * (1.0 / n)

    return pl.pallas_call(
        body, grid=(n // nblk,),
        in_specs=[pl.BlockSpec((nblk, f), lambda i: (i, 0))],
        out_specs=pl.BlockSpec((1, f), lambda i: (0, 0)),
        out_shape=jax.ShapeDtypeStruct((1, f), jnp.float32))(x)


def _mlp(mx, m1, m2, m3, w1, b1, w2, b2, w3, b3):
    f0, f1, f2, f3 = mx.shape[1], m1.shape[1], m2.shape[1], m3.shape[1]

    def body(mx_r, m1_r, m2_r, m3_r, w1_r, b1_r, w2_r, b2_r, w3_r, b3_r, o):
        h = (jnp.dot(mx_r[...], w1_r[pl.ds(0, f0), :],
                     preferred_element_type=jnp.float32)
             + jnp.dot(m1_r[...], w1_r[pl.ds(f0, f1), :],
                       preferred_element_type=jnp.float32)
             + jnp.dot(m2_r[...], w1_r[pl.ds(f0 + f1, f2), :],
                       preferred_element_type=jnp.float32)
             + jnp.dot(m3_r[...], w1_r[pl.ds(f0 + f1 + f2, f3), :],
                       preferred_element_type=jnp.float32)
             + b1_r[...])
        h = jnp.maximum(h, 0.0)
        h2 = jnp.maximum(
            jnp.dot(h, w2_r[...], preferred_element_type=jnp.float32)
            + b2_r[...], 0.0)
        o[...] = (jnp.dot(h2, w3_r[...], preferred_element_type=jnp.float32)
                  + b3_r[...])

    nout = b3.shape[0]
    return pl.pallas_call(
        body,
        out_shape=jax.ShapeDtypeStruct((1, nout), jnp.float32),
    )(mx, m1, m2, m3, w1, b1.reshape(1, -1), w2, b2.reshape(1, -1),
      w3, b3.reshape(1, -1))


def _gat_layer(xs, wl, wr, att, b, g, be, rm, rv, idx, n, c):
    f_tot = wl.shape[0]
    wlh = wl.reshape(f_tot, H, c).transpose(1, 0, 2)
    wrh = wr.reshape(f_tot, H, c).transpose(1, 0, 2)
    xl, xr = _proj_heads(xs, wlh, wrh, c)
    acc = _gat_edge_sc(xl, xr, idx, att, n, c)
    return _combine(acc, b, g, be, rm, rv, n, c).reshape(n, H * c)


def kernel(x, edge_index, Wl1, Wr1, att1, b1, Wl2, Wr2, att2, b2,
           Wl3, Wr3, att3, b3, g1, be1, rm1, rv1, g2, be2, rm2, rv2,
           g3, be3, rm3, rv3, Wm1, bm1, Wm2, bm2, Wm3, bm3):
    n = x.shape[0]
    e = edge_index.shape[1]
    loop = jnp.arange(n, dtype=jnp.int32)
    src = jnp.concatenate([edge_index[0].astype(jnp.int32), loop])
    dst = jnp.concatenate([edge_index[1].astype(jnp.int32), loop])
    etot = e + n
    quant = NUM_TILES * K_EDGES * 2
    epad = -(-etot // quant) * quant
    pad = epad - etot
    nchunk = epad // (NUM_TILES * K_EDGES)
    shp = (NUM_TILES, nchunk, K_EDGES)
    idx_p = jnp.concatenate(
        [jnp.pad(src, (0, pad)).reshape(shp),
         jnp.pad(dst, (0, pad)).reshape(shp),
         jnp.pad(dst, (0, pad), constant_values=n).reshape(shp)], axis=2)

    x1 = _gat_layer([x], Wl1, Wr1, att1, b1, g1, be1, rm1, rv1,
                    idx_p, n, 128)
    x2 = _gat_layer([x1], Wl2, Wr2, att2, b2, g2, be2, rm2, rv2,
                    idx_p, n, 64)
    x3 = _gat_layer([x, x2], Wl3, Wr3, att3, b3, g3, be3, rm3, rv3,
                    idx_p, n, 32)

    return _mlp(_colmean(x), _colmean(x1), _colmean(x2), _colmean(x3),
                Wm1, bm1, Wm2, bm2, Wm3, bm3)


# speedup vs baseline: 3.6085x; 1.0775x over previous
import functools

import jax
import jax.numpy as jnp
from jax import lax
from jax.experimental import pallas as pl
from jax.experimental.pallas import tpu as pltpu
from jax.experimental.pallas import tpu_sc as plsc

H = 4
K_EDGES = 32
NUM_TILES = 32


def _lane_take(v, idx16):
    dnums = lax.GatherDimensionNumbers(
        offset_dims=(), collapsed_slice_dims=(0,), start_index_map=(0,))
    return lax.gather(v, idx16[:, None], dnums, (1,),
                      mode=lax.GatherScatterMode.PROMISE_IN_BOUNDS)


def _largest_div(n, cap):
    for d in range(min(n, cap), 0, -1):
        if n % d == 0:
            return d
    return 1


def _gat_edge_sc(xl, xr, idx, att, n, c):
    cp = c + 16
    nchunk = idx.shape[1]
    assert idx.shape == (NUM_TILES, nchunk, 3 * K_EDGES) and nchunk % 2 == 0
    npt = -(-(-(-n // 16)) // 128) * 128
    while 16 * npt <= n:
        npt += 128
    n_pad = 16 * npt
    zr = K_EDGES
    assert npt % zr == 0
    nz = npt // zr
    cblk = c // 16

    mesh = plsc.VectorSubcoreMesh(core_axis_name="c", subcore_axis_name="s",
                                  num_cores=2, num_subcores=16)

    @functools.partial(
        pl.kernel,
        out_type=jax.ShapeDtypeStruct((2, H, n_pad, cp), jnp.float32),
        mesh=mesh,
        scratch_types=[
            pltpu.VMEM((nchunk, 3 * K_EDGES), jnp.int32),
            [pltpu.VMEM((K_EDGES,), jnp.int32)] * 2,
            [pltpu.VMEM((K_EDGES,), jnp.int32)] * 2,
            [pltpu.VMEM((K_EDGES,), jnp.int32)] * 2,
            [pltpu.VMEM((K_EDGES, c), jnp.float32)] * 2,
            [pltpu.VMEM((K_EDGES, c), jnp.float32)] * 2,
            [pltpu.VMEM((K_EDGES, cp), jnp.float32)] * 2,
            pltpu.VMEM((c,), jnp.float32),
            pltpu.VMEM_SHARED((n_pad, cp), jnp.float32),
            [pltpu.SemaphoreType.DMA] * 2,
            [pltpu.SemaphoreType.DMA] * 2,
        ],
        compiler_params=pltpu.CompilerParams(needs_layout_passes=False,
                                             use_tc_tiling_on_sc=False),
    )
    def k(xl_hbm, xr_hbm, idx_hbm, att_hbm, out_hbm,
          idxall, srchv, dsthv, dscv, rl, rr, sbuf,
          attv, acc, gsem, ssem):
        core = lax.axis_index("c")
        sub = lax.axis_index("s")
        tid = core * 16 + sub
        row0 = sub * npt

        pltpu.sync_copy(idx_hbm.at[tid], idxall)

        z16 = jnp.zeros((16,), jnp.float32)

        @pl.loop(0, H)
        def _head(h):
            @pl.loop(0, K_EDGES)
            def _(i):
                for cb in range(cp // 16):
                    sbuf[0][i, pl.ds(cb * 16, 16)] = z16

            for j in range(nz):
                pltpu.sync_copy(sbuf[0], acc.at[pl.ds(row0 + j * zr, zr)])
            pltpu.sync_copy(att_hbm.at[h], attv)
            att_b = [attv[pl.ds(cb * 16, 16)] for cb in range(cblk)]
            hn = h * n

            def load_idx(g, b):
                for j in range(K_EDGES // 16):
                    srchv[b][pl.ds(j * 16, 16)] = (
                        idxall[g, pl.ds(j * 16, 16)] + hn)
                    dsthv[b][pl.ds(j * 16, 16)] = (
                        idxall[g, pl.ds(K_EDGES + j * 16, 16)] + hn)

            def start_gather(b):
                pltpu.async_copy(xl_hbm.at[srchv[b]], rl[b], gsem[b])
                pltpu.async_copy(xr_hbm.at[dsthv[b]], rr[b], gsem[b])

            def wait_gather(b):
                pltpu.make_async_copy(xl_hbm.at[srchv[b]], rl[b],
                                      gsem[b]).wait()
                pltpu.make_async_copy(xr_hbm.at[dsthv[b]], rr[b],
                                      gsem[b]).wait()

            def wait_scatter(b):
                pltpu.make_async_copy(sbuf[b], acc.at[dscv[b]],
                                      ssem[b]).wait()

            def compute(g, b):
                for j in range(K_EDGES // 16):
                    dscv[b][pl.ds(j * 16, 16)] = idxall[
                        g, pl.ds(2 * K_EDGES + j * 16, 16)]
                lane = lax.iota(jnp.int32, 16)
                for i in range(K_EDGES):
                    accv = None
                    for cb in range(cblk):
                        sl = pl.ds(cb * 16, 16)
                        s = rl[b][i, sl] + rr[b][i, sl]
                        lrel = jnp.maximum(s, 0.2 * s)
                        t = lrel * att_b[cb]
                        accv = t if accv is None else accv + t
                    ex = jnp.exp(jnp.full((16,), jnp.sum(accv)))
                    for cb in range(cblk):
                        sl = pl.ds(cb * 16, 16)
                        sbuf[b][i, sl] = rl[b][i, sl] * ex
                    sbuf[b][i, pl.ds(c, 16)] = jnp.where(lane == 0, ex, 0.0)
                pltpu.async_copy(sbuf[b], acc.at[dscv[b]], ssem[b],
                                 add=True)

            plsc.subcore_barrier()

            load_idx(0, 0)
            start_gather(0)

            @pl.loop(0, nchunk, step=2)
            def _chunk(g):
                load_idx(g + 1, 1)
                start_gather(1)
                wait_gather(0)

                @pl.when(g >= 2)
                def _():
                    wait_scatter(0)
                compute(g, 0)

                @pl.when(g + 2 < nchunk)
                def _():
                    load_idx(g + 2, 0)
                    start_gather(0)
                wait_gather(1)

                @pl.when(g >= 2)
                def _():
                    wait_scatter(1)
                compute(g + 1, 1)

            wait_scatter(0)
            wait_scatter(1)
            plsc.subcore_barrier()
            pltpu.sync_copy(acc.at[pl.ds(row0, npt)],
                            out_hbm.at[core, h, pl.ds(row0, npt)])
            plsc.subcore_barrier()

    return k(xl, xr, idx, att)


def _proj_heads(xs, wlh, wrh, c, part=None):
    n = xs[0].shape[0]
    nblk = _largest_div(n, 1024)
    f_tot = wlh.shape[1]
    splits = [x.shape[1] for x in xs]
    np_ = 2 if part is not None else 0

    def body(*refs):
        xrefs = refs[:len(xs)]
        prefs = refs[len(xs):len(xs) + np_]
        wl_ref, wr_ref, xl_ref, xr_ref = refs[len(xs) + np_:]
        for h in range(H):
            accl = prefs[0][h] if np_ else None
            accr = prefs[1][h] if np_ else None
            off = 0
            for xi, fi in zip(xrefs, splits):
                xb = xi[...]
                pl_w = wl_ref[h, pl.ds(off, fi), :]
                pr_w = wr_ref[h, pl.ds(off, fi), :]
                dl = jnp.dot(xb, pl_w, preferred_element_type=jnp.float32)
                dr = jnp.dot(xb, pr_w, preferred_element_type=jnp.float32)
                accl = dl if accl is None else accl + dl
                accr = dr if accr is None else accr + dr
                off += fi
            xl_ref[h] = accl
            xr_ref[h] = accr

    grid = (n // nblk,)
    in_specs = [pl.BlockSpec((nblk, fi), lambda i: (i, 0)) for fi in splits]
    in_specs += [pl.BlockSpec((H, nblk, c), lambda i: (0, i, 0))] * np_
    in_specs += [pl.BlockSpec((H, f_tot, c), lambda i: (0, 0, 0))] * 2
    out_specs = [pl.BlockSpec((H, nblk, c), lambda i: (0, i, 0))] * 2
    out_shape = [jax.ShapeDtypeStruct((H, n, c), jnp.float32)] * 2
    args = [x.astype(jnp.bfloat16) for x in xs]
    args += list(part) if part is not None else []
    args += [wlh.astype(jnp.bfloat16), wrh.astype(jnp.bfloat16)]
    return pl.pallas_call(
        body, grid=grid, in_specs=in_specs, out_specs=out_specs,
        out_shape=out_shape)(*args)


def _combine(acc, b, g, be, rm, rv, n, c):
    cp = c + 16
    nblk = _largest_div(n, 1024)

    def body(acc_ref, b_ref, g_ref, be_ref, rm_ref, rv_ref, o_ref, m_ref):
        @pl.when(pl.program_id(0) == 0)
        def _():
            m_ref[...] = jnp.zeros_like(m_ref)
        for h in range(H):
            a = acc_ref[0, h] + acc_ref[1, h]
            num = a[:, :c]
            den = a[:, c:c + 1]
            v = num / (den + 1e-16) + b_ref[h]
            v = jnp.maximum(v, 0.0)
            v = (v - rm_ref[h]) * jax.lax.rsqrt(rv_ref[h] + 1e-5)
            v = v * g_ref[h] + be_ref[h]
            o_ref[:, h, :] = v
            m_ref[0, h, :] += jnp.sum(v, axis=0) * (1.0 / n)

    grid = (n // nblk,)
    vec = pl.BlockSpec((H, 1, c), lambda i: (0, 0, 0))
    r3 = lambda a: a.reshape(H, 1, c)
    return pl.pallas_call(
        body, grid=grid,
        in_specs=[pl.BlockSpec((2, H, nblk, cp), lambda i: (0, 0, i, 0)),
                  vec, vec, vec, vec, vec],
        out_specs=[pl.BlockSpec((nblk, H, c), lambda i: (i, 0, 0)),
                   pl.BlockSpec((1, H, c), lambda i: (0, 0, 0))],
        out_shape=[jax.ShapeDtypeStruct((n, H, c), jnp.float32),
                   jax.ShapeDtypeStruct((1, H, c), jnp.float32)],
    )(acc, r3(b), r3(g), r3(be), r3(rm), r3(rv))


def _colmean(x):
    n, f = x.shape
    nblk = _largest_div(n, 1024)

    def body(x_ref, o_ref):
        @pl.when(pl.program_id(0) == 0)
        def _():
            o_ref[...] = jnp.zeros_like(o_ref)
        o_ref[...] += jnp.sum(x_ref[...], axis=0, keepdims=True) * (1.0 / n)

    return pl.pallas_call(
        body, grid=(n // nblk,),
        in_specs=[pl.BlockSpec((nblk, f), lambda i: (i, 0))],
        out_specs=pl.BlockSpec((1, f), lambda i: (0, 0)),
        out_shape=jax.ShapeDtypeStruct((1, f), jnp.float32))(x)


def _mlp(mx, m1, m2, m3, w1, b1, w2, b2, w3, b3):
    f0, f1, f2, f3 = mx.shape[1], m1.shape[1], m2.shape[1], m3.shape[1]

    def body(mx_r, m1_r, m2_r, m3_r, w1_r, b1_r, w2_r, b2_r, w3_r, b3_r, o):
        h = (jnp.dot(mx_r[...], w1_r[pl.ds(0, f0), :],
                     preferred_element_type=jnp.float32)
             + jnp.dot(m1_r[...], w1_r[pl.ds(f0, f1), :],
                       preferred_element_type=jnp.float32)
             + jnp.dot(m2_r[...], w1_r[pl.ds(f0 + f1, f2), :],
                       preferred_element_type=jnp.float32)
             + jnp.dot(m3_r[...], w1_r[pl.ds(f0 + f1 + f2, f3), :],
                       preferred_element_type=jnp.float32)
             + b1_r[...])
        h = jnp.maximum(h, 0.0)
        h2 = jnp.maximum(
            jnp.dot(h, w2_r[...], preferred_element_type=jnp.float32)
            + b2_r[...], 0.0)
        o[...] = (jnp.dot(h2, w3_r[...], preferred_element_type=jnp.float32)
                  + b3_r[...])

    nout = b3.shape[0]
    return pl.pallas_call(
        body,
        out_shape=jax.ShapeDtypeStruct((1, nout), jnp.float32),
    )(mx, m1, m2, m3, w1, b1.reshape(1, -1), w2, b2.reshape(1, -1),
      w3, b3.reshape(1, -1))


def _gat_layer(xs, wlh, wrh, att, b, g, be, rm, rv, idx, n, c, part=None):
    xl, xr = _proj_heads(xs, wlh, wrh, c, part)
    acc = _gat_edge_sc(xl.reshape(H * n, c), xr.reshape(H * n, c),
                       idx, att, n, c)
    xn, m = _combine(acc, b, g, be, rm, rv, n, c)
    return xn.reshape(n, H * c), m.reshape(1, H * c)


def kernel(x, edge_index, Wl1, Wr1, att1, b1, Wl2, Wr2, att2, b2,
           Wl3, Wr3, att3, b3, g1, be1, rm1, rv1, g2, be2, rm2, rv2,
           g3, be3, rm3, rv3, Wm1, bm1, Wm2, bm2, Wm3, bm3):
    n = x.shape[0]
    e = edge_index.shape[1]
    loop = jnp.arange(n, dtype=jnp.int32)
    src = jnp.concatenate([edge_index[0].astype(jnp.int32), loop])
    dst = jnp.concatenate([edge_index[1].astype(jnp.int32), loop])
    etot = e + n
    quant = NUM_TILES * K_EDGES * 2
    epad = -(-etot // quant) * quant
    pad = epad - etot
    nchunk = epad // (NUM_TILES * K_EDGES)
    shp = (NUM_TILES, nchunk, K_EDGES)
    idx_p = jnp.concatenate(
        [jnp.pad(src, (0, pad)).reshape(shp),
         jnp.pad(dst, (0, pad)).reshape(shp),
         jnp.pad(dst, (0, pad), constant_values=n).reshape(shp)], axis=2)

    def _mk(w, c):
        return w.reshape(w.shape[0], H, c).transpose(1, 0, 2)

    wlh1, wrh1 = _mk(Wl1, 128), _mk(Wr1, 128)
    wlh2, wrh2 = _mk(Wl2, 64), _mk(Wr2, 64)
    wlh3, wrh3 = _mk(Wl3, 32), _mk(Wr3, 32)
    f0 = x.shape[1]

    part3 = _proj_heads([x], wlh3[:, :f0], wrh3[:, :f0], 32)
    x1, m1 = _gat_layer([x], wlh1, wrh1, att1, b1, g1, be1, rm1, rv1,
                        idx_p, n, 128)
    x2, m2 = _gat_layer([x1], wlh2, wrh2, att2, b2, g2, be2, rm2, rv2,
                        idx_p, n, 64)
    _, m3 = _gat_layer([x2], wlh3[:, f0:], wrh3[:, f0:], att3, b3,
                       g3, be3, rm3, rv3, idx_p, n, 32, part=part3)

    return _mlp(_colmean(x), m1, m2, m3, Wm1, bm1, Wm2, bm2, Wm3, bm3)


# speedup vs baseline: 3.6540x; 1.0126x over previous
import functools

import jax
import jax.numpy as jnp
from jax import lax
from jax.experimental import pallas as pl
from jax.experimental.pallas import tpu as pltpu
from jax.experimental.pallas import tpu_sc as plsc

H = 4
K_EDGES = 32
NUM_TILES = 32


def _lane_take(v, idx16):
    dnums = lax.GatherDimensionNumbers(
        offset_dims=(), collapsed_slice_dims=(0,), start_index_map=(0,))
    return lax.gather(v, idx16[:, None], dnums, (1,),
                      mode=lax.GatherScatterMode.PROMISE_IN_BOUNDS)


def _largest_div(n, cap):
    for d in range(min(n, cap), 0, -1):
        if n % d == 0:
            return d
    return 1


def _gat_edge_sc(xl, xr, idx, att, n, c, ke):
    cp = c + 16
    nchunk = idx.shape[1]
    assert idx.shape == (NUM_TILES, nchunk, 3 * ke) and nchunk % 2 == 0
    npt = -(-(-(-n // 16)) // 128) * 128
    while 16 * npt <= n:
        npt += 128
    n_pad = 16 * npt
    zr = ke
    assert npt % zr == 0
    nz = npt // zr
    cblk = c // 16

    mesh = plsc.VectorSubcoreMesh(core_axis_name="c", subcore_axis_name="s",
                                  num_cores=2, num_subcores=16)

    @functools.partial(
        pl.kernel,
        out_type=jax.ShapeDtypeStruct((2, H, n_pad, cp), jnp.float32),
        mesh=mesh,
        scratch_types=[
            pltpu.VMEM((nchunk, 3 * ke), jnp.int32),
            [pltpu.VMEM((ke,), jnp.int32)] * 2,
            [pltpu.VMEM((ke,), jnp.int32)] * 2,
            [pltpu.VMEM((ke,), jnp.int32)] * 2,
            [pltpu.VMEM((ke, c), jnp.float32)] * 2,
            [pltpu.VMEM((ke, c), jnp.float32)] * 2,
            [pltpu.VMEM((ke, cp), jnp.float32)] * 2,
            pltpu.VMEM((c,), jnp.float32),
            pltpu.VMEM_SHARED((n_pad, cp), jnp.float32),
            [pltpu.SemaphoreType.DMA] * 2,
            [pltpu.SemaphoreType.DMA] * 2,
        ],
        compiler_params=pltpu.CompilerParams(needs_layout_passes=False,
                                             use_tc_tiling_on_sc=False),
    )
    def k(xl_hbm, xr_hbm, idx_hbm, att_hbm, out_hbm,
          idxall, srchv, dsthv, dscv, rl, rr, sbuf,
          attv, acc, gsem, ssem):
        core = lax.axis_index("c")
        sub = lax.axis_index("s")
        tid = core * 16 + sub
        row0 = sub * npt

        pltpu.sync_copy(idx_hbm.at[tid], idxall)

        z16 = jnp.zeros((16,), jnp.float32)

        @pl.loop(0, H)
        def _head(h):
            @pl.loop(0, ke)
            def _(i):
                for cb in range(cp // 16):
                    sbuf[0][i, pl.ds(cb * 16, 16)] = z16

            for j in range(nz):
                pltpu.sync_copy(sbuf[0], acc.at[pl.ds(row0 + j * zr, zr)])
            pltpu.sync_copy(att_hbm.at[h], attv)
            att_b = [attv[pl.ds(cb * 16, 16)] for cb in range(cblk)]
            hn = h * n

            def load_idx(g, b):
                for j in range(ke // 16):
                    srchv[b][pl.ds(j * 16, 16)] = (
                        idxall[g, pl.ds(j * 16, 16)] + hn)
                    dsthv[b][pl.ds(j * 16, 16)] = (
                        idxall[g, pl.ds(ke + j * 16, 16)] + hn)

            def start_gather(b):
                pltpu.async_copy(xl_hbm.at[srchv[b]], rl[b], gsem[b])
                pltpu.async_copy(xr_hbm.at[dsthv[b]], rr[b], gsem[b])

            def wait_gather(b):
                pltpu.make_async_copy(xl_hbm.at[srchv[b]], rl[b],
                                      gsem[b]).wait()
                pltpu.make_async_copy(xr_hbm.at[dsthv[b]], rr[b],
                                      gsem[b]).wait()

            def wait_scatter(b):
                pltpu.make_async_copy(sbuf[b], acc.at[dscv[b]],
                                      ssem[b]).wait()

            def compute(g, b):
                for j in range(ke // 16):
                    dscv[b][pl.ds(j * 16, 16)] = idxall[
                        g, pl.ds(2 * ke + j * 16, 16)]
                lane = lax.iota(jnp.int32, 16)
                for i in range(ke):
                    accv = None
                    for cb in range(cblk):
                        sl = pl.ds(cb * 16, 16)
                        s = rl[b][i, sl] + rr[b][i, sl]
                        lrel = jnp.maximum(s, 0.2 * s)
                        t = lrel * att_b[cb]
                        accv = t if accv is None else accv + t
                    ex = jnp.exp(jnp.full((16,), jnp.sum(accv)))
                    for cb in range(cblk):
                        sl = pl.ds(cb * 16, 16)
                        sbuf[b][i, sl] = rl[b][i, sl] * ex
                    sbuf[b][i, pl.ds(c, 16)] = jnp.where(lane == 0, ex, 0.0)
                pltpu.async_copy(sbuf[b], acc.at[dscv[b]], ssem[b],
                                 add=True)

            plsc.subcore_barrier()

            load_idx(0, 0)
            start_gather(0)

            @pl.loop(0, nchunk, step=2)
            def _chunk(g):
                load_idx(g + 1, 1)
                start_gather(1)
                wait_gather(0)

                @pl.when(g >= 2)
                def _():
                    wait_scatter(0)
                compute(g, 0)

                @pl.when(g + 2 < nchunk)
                def _():
                    load_idx(g + 2, 0)
                    start_gather(0)
                wait_gather(1)

                @pl.when(g >= 2)
                def _():
                    wait_scatter(1)
                compute(g + 1, 1)

            wait_scatter(0)
            wait_scatter(1)
            plsc.subcore_barrier()
            pltpu.sync_copy(acc.at[pl.ds(row0, npt)],
                            out_hbm.at[core, h, pl.ds(row0, npt)])
            plsc.subcore_barrier()

    return k(xl, xr, idx, att)


def _proj_heads(xs, wlh, wrh, c, part=None):
    n = xs[0].shape[0]
    nblk = _largest_div(n, 1024)
    f_tot = wlh.shape[1]
    splits = [x.shape[1] for x in xs]
    np_ = 2 if part is not None else 0

    def body(*refs):
        xrefs = refs[:len(xs)]
        prefs = refs[len(xs):len(xs) + np_]
        wl_ref, wr_ref, xl_ref, xr_ref = refs[len(xs) + np_:]
        for h in range(H):
            accl = prefs[0][h] if np_ else None
            accr = prefs[1][h] if np_ else None
            off = 0
            for xi, fi in zip(xrefs, splits):
                xb = xi[...]
                pl_w = wl_ref[h, pl.ds(off, fi), :]
                pr_w = wr_ref[h, pl.ds(off, fi), :]
                dl = jnp.dot(xb, pl_w, preferred_element_type=jnp.float32)
                dr = jnp.dot(xb, pr_w, preferred_element_type=jnp.float32)
                accl = dl if accl is None else accl + dl
                accr = dr if accr is None else accr + dr
                off += fi
            xl_ref[h] = accl
            xr_ref[h] = accr

    grid = (n // nblk,)
    in_specs = [pl.BlockSpec((nblk, fi), lambda i: (i, 0)) for fi in splits]
    in_specs += [pl.BlockSpec((H, nblk, c), lambda i: (0, i, 0))] * np_
    in_specs += [pl.BlockSpec((H, f_tot, c), lambda i: (0, 0, 0))] * 2
    out_specs = [pl.BlockSpec((H, nblk, c), lambda i: (0, i, 0))] * 2
    out_shape = [jax.ShapeDtypeStruct((H, n, c), jnp.float32)] * 2
    args = [x.astype(jnp.bfloat16) for x in xs]
    args += list(part) if part is not None else []
    args += [wlh.astype(jnp.bfloat16), wrh.astype(jnp.bfloat16)]
    return pl.pallas_call(
        body, grid=grid, in_specs=in_specs, out_specs=out_specs,
        out_shape=out_shape)(*args)


def _combine(acc, b, g, be, rm, rv, n, c):
    cp = c + 16
    nblk = _largest_div(n, 1024)

    def body(acc_ref, b_ref, g_ref, be_ref, rm_ref, rv_ref, o_ref, m_ref):
        @pl.when(pl.program_id(0) == 0)
        def _():
            m_ref[...] = jnp.zeros_like(m_ref)
        for h in range(H):
            a = acc_ref[0, h] + acc_ref[1, h]
            num = a[:, :c]
            den = a[:, c:c + 1]
            v = num / (den + 1e-16) + b_ref[h]
            v = jnp.maximum(v, 0.0)
            v = (v - rm_ref[h]) * jax.lax.rsqrt(rv_ref[h] + 1e-5)
            v = v * g_ref[h] + be_ref[h]
            o_ref[:, h, :] = v
            m_ref[0, h, :] += jnp.sum(v, axis=0) * (1.0 / n)

    grid = (n // nblk,)
    vec = pl.BlockSpec((H, 1, c), lambda i: (0, 0, 0))
    r3 = lambda a: a.reshape(H, 1, c)
    return pl.pallas_call(
        body, grid=grid,
        in_specs=[pl.BlockSpec((2, H, nblk, cp), lambda i: (0, 0, i, 0)),
                  vec, vec, vec, vec, vec],
        out_specs=[pl.BlockSpec((nblk, H, c), lambda i: (i, 0, 0)),
                   pl.BlockSpec((1, H, c), lambda i: (0, 0, 0))],
        out_shape=[jax.ShapeDtypeStruct((n, H, c), jnp.float32),
                   jax.ShapeDtypeStruct((1, H, c), jnp.float32)],
    )(acc, r3(b), r3(g), r3(be), r3(rm), r3(rv))


def _colmean(x):
    n, f = x.shape
    nblk = _largest_div(n, 1024)

    def body(x_ref, o_ref):
        @pl.when(pl.program_id(0) == 0)
        def _():
            o_ref[...] = jnp.zeros_like(o_ref)
        o_ref[...] += jnp.sum(x_ref[...], axis=0, keepdims=True) * (1.0 / n)

    return pl.pallas_call(
        body, grid=(n // nblk,),
        in_specs=[pl.BlockSpec((nblk, f), lambda i: (i, 0))],
        out_specs=pl.BlockSpec((1, f), lambda i: (0, 0)),
        out_shape=jax.ShapeDtypeStruct((1, f), jnp.float32))(x)


def _mlp(mx, m1, m2, m3, w1, b1, w2, b2, w3, b3):
    f0, f1, f2, f3 = mx.shape[1], m1.shape[1], m2.shape[1], m3.shape[1]

    def body(mx_r, m1_r, m2_r, m3_r, w1_r, b1_r, w2_r, b2_r, w3_r, b3_r, o):
        h = (jnp.dot(mx_r[...], w1_r[pl.ds(0, f0), :],
                     preferred_element_type=jnp.float32)
             + jnp.dot(m1_r[...], w1_r[pl.ds(f0, f1), :],
                       preferred_element_type=jnp.float32)
             + jnp.dot(m2_r[...], w1_r[pl.ds(f0 + f1, f2), :],
                       preferred_element_type=jnp.float32)
             + jnp.dot(m3_r[...], w1_r[pl.ds(f0 + f1 + f2, f3), :],
                       preferred_element_type=jnp.float32)
             + b1_r[...])
        h = jnp.maximum(h, 0.0)
        h2 = jnp.maximum(
            jnp.dot(h, w2_r[...], preferred_element_type=jnp.float32)
            + b2_r[...], 0.0)
        o[...] = (jnp.dot(h2, w3_r[...], preferred_element_type=jnp.float32)
                  + b3_r[...])

    nout = b3.shape[0]
    return pl.pallas_call(
        body,
        out_shape=jax.ShapeDtypeStruct((1, nout), jnp.float32),
    )(mx, m1, m2, m3, w1, b1.reshape(1, -1), w2, b2.reshape(1, -1),
      w3, b3.reshape(1, -1))


def _gat_layer(xs, wlh, wrh, att, b, g, be, rm, rv, idx, n, c, ke,
               part=None):
    xl, xr = _proj_heads(xs, wlh, wrh, c, part)
    acc = _gat_edge_sc(xl.reshape(H * n, c), xr.reshape(H * n, c),
                       idx, att, n, c, ke)
    xn, m = _combine(acc, b, g, be, rm, rv, n, c)
    return xn.reshape(n, H * c), m.reshape(1, H * c)


def kernel(x, edge_index, Wl1, Wr1, att1, b1, Wl2, Wr2, att2, b2,
           Wl3, Wr3, att3, b3, g1, be1, rm1, rv1, g2, be2, rm2, rv2,
           g3, be3, rm3, rv3, Wm1, bm1, Wm2, bm2, Wm3, bm3):
    n = x.shape[0]
    e = edge_index.shape[1]
    loop = jnp.arange(n, dtype=jnp.int32)
    src = jnp.concatenate([edge_index[0].astype(jnp.int32), loop])
    dst = jnp.concatenate([edge_index[1].astype(jnp.int32), loop])
    etot = e + n

    def _mk_idx(ke):
        quant = NUM_TILES * ke * 2
        epad = -(-etot // quant) * quant
        pad = epad - etot
        shp = (NUM_TILES, epad // (NUM_TILES * ke), ke)
        return jnp.concatenate(
            [jnp.pad(src, (0, pad)).reshape(shp),
             jnp.pad(dst, (0, pad)).reshape(shp),
             jnp.pad(dst, (0, pad), constant_values=n).reshape(shp)], axis=2)

    idx32 = _mk_idx(32)
    idx64 = _mk_idx(64)

    def _mk(w, c):
        return w.reshape(w.shape[0], H, c).transpose(1, 0, 2)

    wlh1, wrh1 = _mk(Wl1, 128), _mk(Wr1, 128)
    wlh2, wrh2 = _mk(Wl2, 64), _mk(Wr2, 64)
    wlh3, wrh3 = _mk(Wl3, 32), _mk(Wr3, 32)
    f0 = x.shape[1]

    part3 = _proj_heads([x], wlh3[:, :f0], wrh3[:, :f0], 32)
    x1, m1 = _gat_layer([x], wlh1, wrh1, att1, b1, g1, be1, rm1, rv1,
                        idx32, n, 128, 32)
    x2, m2 = _gat_layer([x1], wlh2, wrh2, att2, b2, g2, be2, rm2, rv2,
                        idx64, n, 64, 64)
    _, m3 = _gat_layer([x2], wlh3[:, f0:], wrh3[:, f0:], att3, b3,
                       g3, be3, rm3, rv3, idx64, n, 32, 64, part=part3)

    return _mlp(_colmean(x), m1, m2, m3, Wm1, bm1, Wm2, bm2, Wm3, bm3)


# speedup vs baseline: 3.6648x; 1.0030x over previous
import functools

import jax
import jax.numpy as jnp
from jax import lax
from jax.experimental import pallas as pl
from jax.experimental.pallas import tpu as pltpu
from jax.experimental.pallas import tpu_sc as plsc

H = 4
K_EDGES = 32
NUM_TILES = 32


def _largest_div(n, cap):
    for d in range(min(n, cap), 0, -1):
        if n % d == 0:
            return d
    return 1


def _gat_edge_sc(xl, xr, idx, att, n, c, ke):
    cp = c + 16
    nchunk = idx.shape[1]
    assert idx.shape == (NUM_TILES, nchunk, 3 * ke) and nchunk % 2 == 0
    npt = -(-(-(-n // 16)) // 128) * 128
    while 16 * npt <= n:
        npt += 128
    n_pad = 16 * npt
    zr = ke
    assert npt % zr == 0
    nz = npt // zr
    cblk = c // 16

    mesh = plsc.VectorSubcoreMesh(core_axis_name="c", subcore_axis_name="s",
                                  num_cores=2, num_subcores=16)

    @functools.partial(
        pl.kernel,
        out_type=jax.ShapeDtypeStruct((2, H, n_pad, cp), jnp.float32),
        mesh=mesh,
        scratch_types=[
            pltpu.VMEM((nchunk, 3 * ke), jnp.int32),
            [pltpu.VMEM((ke,), jnp.int32)] * 2,
            [pltpu.VMEM((ke,), jnp.int32)] * 2,
            [pltpu.VMEM((ke,), jnp.int32)] * 2,
            [pltpu.VMEM((ke, c), jnp.float32)] * 2,
            [pltpu.VMEM((ke, c), jnp.float32)] * 2,
            [pltpu.VMEM((ke, cp), jnp.float32)] * 2,
            pltpu.VMEM((c,), jnp.float32),
            pltpu.VMEM_SHARED((n_pad, cp), jnp.float32),
            [pltpu.SemaphoreType.DMA] * 2,
            [pltpu.SemaphoreType.DMA] * 2,
        ],
        compiler_params=pltpu.CompilerParams(needs_layout_passes=False,
                                             use_tc_tiling_on_sc=False),
    )
    def k(xl_hbm, xr_hbm, idx_hbm, att_hbm, out_hbm,
          idxall, srchv, dsthv, dscv, rl, rr, sbuf,
          attv, acc, gsem, ssem):
        core = lax.axis_index("c")
        sub = lax.axis_index("s")
        tid = core * 16 + sub
        row0 = sub * npt

        pltpu.sync_copy(idx_hbm.at[tid], idxall)

        z16 = jnp.zeros((16,), jnp.float32)

        @pl.loop(0, H)
        def _head(h):
            @pl.loop(0, ke)
            def _(i):
                for cb in range(cp // 16):
                    sbuf[0][i, pl.ds(cb * 16, 16)] = z16

            for j in range(nz):
                pltpu.sync_copy(sbuf[0], acc.at[pl.ds(row0 + j * zr, zr)])
            pltpu.sync_copy(att_hbm.at[h], attv)
            att_b = [attv[pl.ds(cb * 16, 16)] for cb in range(cblk)]
            hn = h * n

            def load_idx(g, b):
                for j in range(ke // 16):
                    srchv[b][pl.ds(j * 16, 16)] = (
                        idxall[g, pl.ds(j * 16, 16)] + hn)
                    dsthv[b][pl.ds(j * 16, 16)] = (
                        idxall[g, pl.ds(ke + j * 16, 16)] + hn)

            def start_gather(b):
                pltpu.async_copy(xl_hbm.at[srchv[b]], rl[b], gsem[b])
                pltpu.async_copy(xr_hbm.at[dsthv[b]], rr[b], gsem[b])

            def wait_gather(b):
                pltpu.make_async_copy(xl_hbm.at[srchv[b]], rl[b],
                                      gsem[b]).wait()
                pltpu.make_async_copy(xr_hbm.at[dsthv[b]], rr[b],
                                      gsem[b]).wait()

            def wait_scatter(b):
                pltpu.make_async_copy(sbuf[b], acc.at[dscv[b]],
                                      ssem[b]).wait()

            def compute(g, b):
                for j in range(ke // 16):
                    dscv[b][pl.ds(j * 16, 16)] = idxall[
                        g, pl.ds(2 * ke + j * 16, 16)]
                lane = lax.iota(jnp.int32, 16)
                for i in range(ke):
                    accv = None
                    for cb in range(cblk):
                        sl = pl.ds(cb * 16, 16)
                        s = rl[b][i, sl] + rr[b][i, sl]
                        lrel = jnp.maximum(s, 0.2 * s)
                        t = lrel * att_b[cb]
                        accv = t if accv is None else accv + t
                    ex = jnp.exp(jnp.full((16,), jnp.sum(accv)))
                    for cb in range(cblk):
                        sl = pl.ds(cb * 16, 16)
                        sbuf[b][i, sl] = rl[b][i, sl] * ex
                    sbuf[b][i, pl.ds(c, 16)] = jnp.where(lane == 0, ex, 0.0)
                pltpu.async_copy(sbuf[b], acc.at[dscv[b]], ssem[b],
                                 add=True)

            plsc.subcore_barrier()

            load_idx(0, 0)
            start_gather(0)

            @pl.loop(0, nchunk, step=2)
            def _chunk(g):
                load_idx(g + 1, 1)
                start_gather(1)
                wait_gather(0)

                @pl.when(g >= 2)
                def _():
                    wait_scatter(0)
                compute(g, 0)

                @pl.when(g + 2 < nchunk)
                def _():
                    load_idx(g + 2, 0)
                    start_gather(0)
                wait_gather(1)

                @pl.when(g >= 2)
                def _():
                    wait_scatter(1)
                compute(g + 1, 1)

            wait_scatter(0)
            wait_scatter(1)
            plsc.subcore_barrier()
            pltpu.sync_copy(acc.at[pl.ds(row0, npt)],
                            out_hbm.at[core, h, pl.ds(row0, npt)])
            plsc.subcore_barrier()

    return k(xl, xr, idx, att)


def _proj_heads(xs, wlh, wrh, c, part=None):
    n = xs[0].shape[0]
    nblk = _largest_div(n, 1024)
    f_tot = wlh.shape[1]
    splits = [x.shape[1] for x in xs]
    np_ = 2 if part is not None else 0

    def body(*refs):
        xrefs = refs[:len(xs)]
        prefs = refs[len(xs):len(xs) + np_]
        wl_ref, wr_ref, xl_ref, xr_ref = refs[len(xs) + np_:]
        for h in range(H):
            accl = prefs[0][h] if np_ else None
            accr = prefs[1][h] if np_ else None
            off = 0
            for xi, fi in zip(xrefs, splits):
                xb = xi[...]
                pl_w = wl_ref[h, pl.ds(off, fi), :]
                pr_w = wr_ref[h, pl.ds(off, fi), :]
                dl = jnp.dot(xb, pl_w, preferred_element_type=jnp.float32)
                dr = jnp.dot(xb, pr_w, preferred_element_type=jnp.float32)
                accl = dl if accl is None else accl + dl
                accr = dr if accr is None else accr + dr
                off += fi
            xl_ref[h] = accl
            xr_ref[h] = accr

    grid = (n // nblk,)
    in_specs = [pl.BlockSpec((nblk, fi), lambda i: (i, 0)) for fi in splits]
    in_specs += [pl.BlockSpec((H, nblk, c), lambda i: (0, i, 0))] * np_
    in_specs += [pl.BlockSpec((H, f_tot, c), lambda i: (0, 0, 0))] * 2
    out_specs = [pl.BlockSpec((H, nblk, c), lambda i: (0, i, 0))] * 2
    out_shape = [jax.ShapeDtypeStruct((H, n, c), jnp.float32)] * 2
    args = [x.astype(jnp.bfloat16) for x in xs]
    args += list(part) if part is not None else []
    args += [wlh.astype(jnp.bfloat16), wrh.astype(jnp.bfloat16)]
    return pl.pallas_call(
        body, grid=grid, in_specs=in_specs, out_specs=out_specs,
        out_shape=out_shape)(*args)


def _combine(acc, b, g, be, rm, rv, n, c):
    cp = c + 16
    nblk = _largest_div(n, 1024)

    def body(acc_ref, b_ref, g_ref, be_ref, rm_ref, rv_ref, o_ref, m_ref):
        @pl.when(pl.program_id(0) == 0)
        def _():
            m_ref[...] = jnp.zeros_like(m_ref)
        for h in range(H):
            a = acc_ref[0, h] + acc_ref[1, h]
            num = a[:, :c]
            den = a[:, c:c + 1]
            v = num / (den + 1e-16) + b_ref[h]
            v = jnp.maximum(v, 0.0)
            v = (v - rm_ref[h]) * jax.lax.rsqrt(rv_ref[h] + 1e-5)
            v = v * g_ref[h] + be_ref[h]
            o_ref[:, h, :] = v
            m_ref[0, h, :] += jnp.sum(v, axis=0) * (1.0 / n)

    grid = (n // nblk,)
    vec = pl.BlockSpec((H, 1, c), lambda i: (0, 0, 0))
    r3 = lambda a: a.reshape(H, 1, c)
    return pl.pallas_call(
        body, grid=grid,
        in_specs=[pl.BlockSpec((2, H, nblk, cp), lambda i: (0, 0, i, 0)),
                  vec, vec, vec, vec, vec],
        out_specs=[pl.BlockSpec((nblk, H, c), lambda i: (i, 0, 0)),
                   pl.BlockSpec((1, H, c), lambda i: (0, 0, 0))],
        out_shape=[jax.ShapeDtypeStruct((n, H, c), jnp.float32),
                   jax.ShapeDtypeStruct((1, H, c), jnp.float32)],
    )(acc, r3(b), r3(g), r3(be), r3(rm), r3(rv))


def _colmean(x):
    n, f = x.shape
    nblk = _largest_div(n, 1024)

    def body(x_ref, o_ref):
        @pl.when(pl.program_id(0) == 0)
        def _():
            o_ref[...] = jnp.zeros_like(o_ref)
        o_ref[...] += jnp.sum(x_ref[...], axis=0, keepdims=True) * (1.0 / n)

    return pl.pallas_call(
        body, grid=(n // nblk,),
        in_specs=[pl.BlockSpec((nblk, f), lambda i: (i, 0))],
        out_specs=pl.BlockSpec((1, f), lambda i: (0, 0)),
        out_shape=jax.ShapeDtypeStruct((1, f), jnp.float32))(x)


def _mlp(mx, m1, m2, m3, w1, b1, w2, b2, w3, b3):
    f0, f1, f2, f3 = mx.shape[1], m1.shape[1], m2.shape[1], m3.shape[1]

    def body(mx_r, m1_r, m2_r, m3_r, w1_r, b1_r, w2_r, b2_r, w3_r, b3_r, o):
        h = (jnp.dot(mx_r[...], w1_r[pl.ds(0, f0), :],
                     preferred_element_type=jnp.float32)
             + jnp.dot(m1_r[...], w1_r[pl.ds(f0, f1), :],
                       preferred_element_type=jnp.float32)
             + jnp.dot(m2_r[...], w1_r[pl.ds(f0 + f1, f2), :],
                       preferred_element_type=jnp.float32)
             + jnp.dot(m3_r[...], w1_r[pl.ds(f0 + f1 + f2, f3), :],
                       preferred_element_type=jnp.float32)
             + b1_r[...])
        h = jnp.maximum(h, 0.0)
        h2 = jnp.maximum(
            jnp.dot(h, w2_r[...], preferred_element_type=jnp.float32)
            + b2_r[...], 0.0)
        o[...] = (jnp.dot(h2, w3_r[...], preferred_element_type=jnp.float32)
                  + b3_r[...])

    nout = b3.shape[0]
    return pl.pallas_call(
        body,
        out_shape=jax.ShapeDtypeStruct((1, nout), jnp.float32),
    )(mx, m1, m2, m3, w1, b1.reshape(1, -1), w2, b2.reshape(1, -1),
      w3, b3.reshape(1, -1))


def _gat_layer(xs, wlh, wrh, att, b, g, be, rm, rv, idx, n, c, ke,
               part=None):
    xl, xr = _proj_heads(xs, wlh, wrh, c, part)
    acc = _gat_edge_sc(xl.reshape(H * n, c), xr.reshape(H * n, c),
                       idx, att, n, c, ke)
    xn, m = _combine(acc, b, g, be, rm, rv, n, c)
    return xn.reshape(n, H * c), m.reshape(1, H * c)


def kernel(x, edge_index, Wl1, Wr1, att1, b1, Wl2, Wr2, att2, b2,
           Wl3, Wr3, att3, b3, g1, be1, rm1, rv1, g2, be2, rm2, rv2,
           g3, be3, rm3, rv3, Wm1, bm1, Wm2, bm2, Wm3, bm3):
    n = x.shape[0]
    e = edge_index.shape[1]
    loop = jnp.arange(n, dtype=jnp.int32)
    src = jnp.concatenate([edge_index[0].astype(jnp.int32), loop])
    dst = jnp.concatenate([edge_index[1].astype(jnp.int32), loop])
    etot = e + n

    def _mk_idx(ke):
        quant = NUM_TILES * ke * 2
        epad = -(-etot // quant) * quant
        pad = epad - etot
        shp = (NUM_TILES, epad // (NUM_TILES * ke), ke)
        return jnp.concatenate(
            [jnp.pad(src, (0, pad)).reshape(shp),
             jnp.pad(dst, (0, pad)).reshape(shp),
             jnp.pad(dst, (0, pad), constant_values=n).reshape(shp)], axis=2)

    idx32 = _mk_idx(32)
    idx64 = _mk_idx(64)

    def _mk(w, c):
        return w.reshape(w.shape[0], H, c).transpose(1, 0, 2)

    wlh1, wrh1 = _mk(Wl1, 128), _mk(Wr1, 128)
    wlh2, wrh2 = _mk(Wl2, 64), _mk(Wr2, 64)
    wlh3, wrh3 = _mk(Wl3, 32), _mk(Wr3, 32)
    f0 = x.shape[1]

    part3 = _proj_heads([x], wlh3[:, :f0], wrh3[:, :f0], 32)
    x1, m1 = _gat_layer([x], wlh1, wrh1, att1, b1, g1, be1, rm1, rv1,
                        idx32, n, 128, 32)
    x2, m2 = _gat_layer([x1], wlh2, wrh2, att2, b2, g2, be2, rm2, rv2,
                        idx64, n, 64, 64)
    _, m3 = _gat_layer([x2], wlh3[:, f0:], wrh3[:, f0:], att3, b3,
                       g3, be3, rm3, rv3, idx64, n, 32, 64, part=part3)

    return _mlp(_colmean(x), m1, m2, m3, Wm1, bm1, Wm2, bm2, Wm3, bm3)
